# baseline (speedup 1.0000x reference)
import functools

import jax
import jax.numpy as jnp
from jax import lax
from jax.experimental import pallas as pl
from jax.experimental.pallas import tpu as pltpu
from jax.experimental.pallas import tpu_sc as plsc

EMBED = 128
ROWS = 17
BATCH = 16384 * 50
NUM_WORKERS = 32
BPW = BATCH // NUM_WORKERS
CHUNK = 256
NCHUNK = BPW // CHUNK
GROUPS = CHUNK // 16


def _lookup(idx_hbm, table_hbm, out_hbm, idx_v, table_v, out_v0, out_v1, wsem):
  wid = lax.axis_index("s") * 2 + lax.axis_index("c")
  base = wid * BPW

  pltpu.sync_copy(table_hbm, table_v)
  pltpu.sync_copy(idx_hbm.at[pl.ds(base, BPW)], idx_v)

  iota = lax.iota(jnp.int32, 16)

  bufs = (out_v0, out_v1)

  def expand(b, chunk):
    out_ref = bufs[b]

    def row_group(g, _):
      idxv = idx_v[pl.ds(chunk * CHUNK + g * 16, 16)]
      src = idxv * EMBED
      dst = (g * 16 + iota) * EMBED
      for _c in range(EMBED):
        val = plsc.load_gather(table_v, [src])
        plsc.store_scatter(out_ref, [dst], val)
        src = src + 1
        dst = dst + 1
      return 0

    lax.fori_loop(0, GROUPS, row_group, 0)

  def start_write(b, chunk):
    off = (base + chunk * CHUNK) * EMBED
    pltpu.async_copy(bufs[b], out_hbm.at[pl.ds(off, CHUNK * EMBED)], wsem)

  def drain_write(b):
    pltpu.make_async_copy(
        bufs[b], out_hbm.at[pl.ds(0, CHUNK * EMBED)], wsem).wait()

  def pair_body(o, _):
    for b in range(2):
      chunk = 2 * o + b
      pl.when(chunk >= 2)(lambda b=b: drain_write(b))
      expand(b, chunk)
      start_write(b, chunk)
    return 0

  lax.fori_loop(0, NCHUNK // 2, pair_body, 0)
  drain_write(0)
  drain_write(1)


def kernel(group_idx, table):
  idx_flat = group_idx.reshape(BATCH)
  table_flat = table.reshape(ROWS * EMBED)
  mesh = plsc.VectorSubcoreMesh(core_axis_name="c", subcore_axis_name="s")
  run = functools.partial(
      pl.kernel,
      out_type=jax.ShapeDtypeStruct((BATCH * EMBED,), jnp.float32),
      mesh=mesh,
      compiler_params=pltpu.CompilerParams(needs_layout_passes=False),
      scratch_types=[
          pltpu.VMEM((BPW,), jnp.int32),
          pltpu.VMEM((ROWS * EMBED,), jnp.float32),
          pltpu.VMEM((CHUNK * EMBED,), jnp.float32),
          pltpu.VMEM((CHUNK * EMBED,), jnp.float32),
          pltpu.SemaphoreType.DMA,
      ],
  )(_lookup)
  out = run(idx_flat, table_flat)
  return out.reshape(group_idx.shape + (EMBED,))


# speedup vs baseline: 4.2181x; 4.2181x over previous
import functools

import jax
import jax.numpy as jnp
from jax import lax
from jax.experimental import pallas as pl
from jax.experimental.pallas import tpu as pltpu
from jax.experimental.pallas import tpu_sc as plsc

EMBED = 128
ROWS = 17
BATCH = 16384 * 50
NUM_WORKERS = 32
BPW = BATCH // NUM_WORKERS
CHUNK = 128
NCHUNK = BPW // CHUNK


def _lookup(idx_hbm, table_hbm, out_hbm, idx_v, table_sh, stage, gsem, wsem):
  sid = lax.axis_index("s")
  wid = sid * 2 + lax.axis_index("c")
  base = wid * BPW

  pl.when(sid == 0)(lambda: pltpu.sync_copy(table_hbm, table_sh))
  plsc.subcore_barrier()
  pltpu.sync_copy(idx_hbm.at[pl.ds(base, BPW)], idx_v)

  def chunk_body(i, _):
    off = base + i * CHUNK
    pltpu.async_copy(
        table_sh.at[idx_v.at[pl.ds(i * CHUNK, CHUNK)]],
        stage,
        gsem,
    ).wait()
    pltpu.sync_copy(stage, out_hbm.at[pl.ds(off, CHUNK)])
    return 0

  lax.fori_loop(0, NCHUNK, chunk_body, 0)


def kernel(group_idx, table):
  idx_flat = group_idx.reshape(BATCH)
  mesh = plsc.VectorSubcoreMesh(core_axis_name="c", subcore_axis_name="s")
  run = functools.partial(
      pl.kernel,
      out_type=jax.ShapeDtypeStruct((BATCH, EMBED), jnp.float32),
      mesh=mesh,
      compiler_params=pltpu.CompilerParams(needs_layout_passes=False),
      scratch_types=[
          pltpu.VMEM((BPW,), jnp.int32),
          pltpu.VMEM_SHARED((ROWS, EMBED), jnp.float32),
          pltpu.VMEM((CHUNK, EMBED), jnp.float32),
          pltpu.SemaphoreType.DMA,
          pltpu.SemaphoreType.DMA,
      ],
  )(_lookup)
  out = run(idx_flat, table)
  return out.reshape(group_idx.shape + (EMBED,))


# speedup vs baseline: 4.6678x; 1.1066x over previous
import functools

import jax
import jax.numpy as jnp
from jax import lax
from jax.experimental import pallas as pl
from jax.experimental.pallas import tpu as pltpu
from jax.experimental.pallas import tpu_sc as plsc

EMBED = 128
ROWS = 17
BATCH = 16384 * 50
NUM_WORKERS = 32
BPW = BATCH // NUM_WORKERS
CHUNK = 128
NCHUNK = BPW // CHUNK


def _lookup(idx_hbm, table_hbm, out_hbm, idx_v, table_sh, stage0, stage1,
            gsem, wsem):
  sid = lax.axis_index("s")
  wid = sid * 2 + lax.axis_index("c")
  base = wid * BPW

  pl.when(sid == 0)(lambda: pltpu.sync_copy(table_hbm, table_sh))
  plsc.subcore_barrier()
  pltpu.sync_copy(idx_hbm.at[pl.ds(base, BPW)], idx_v)

  bufs = (stage0, stage1)

  def gather(b, i):
    pltpu.async_copy(
        table_sh.at[idx_v.at[pl.ds(i * CHUNK, CHUNK)]], bufs[b], gsem).wait()

  def start_write(b, i):
    off = base + i * CHUNK
    pltpu.async_copy(bufs[b], out_hbm.at[pl.ds(off, CHUNK)], wsem)

  def drain_write(b):
    pltpu.make_async_copy(bufs[b], out_hbm.at[pl.ds(0, CHUNK)], wsem).wait()

  def pair_body(o, _):
    for b in range(2):
      i = 2 * o + b
      pl.when(i >= 2)(lambda b=b: drain_write(b))
      gather(b, i)
      start_write(b, i)
    return 0

  lax.fori_loop(0, NCHUNK // 2, pair_body, 0)
  drain_write(0)
  drain_write(1)


def kernel(group_idx, table):
  idx_flat = group_idx.reshape(BATCH)
  mesh = plsc.VectorSubcoreMesh(core_axis_name="c", subcore_axis_name="s")
  run = functools.partial(
      pl.kernel,
      out_type=jax.ShapeDtypeStruct((BATCH, EMBED), jnp.float32),
      mesh=mesh,
      compiler_params=pltpu.CompilerParams(needs_layout_passes=False),
      scratch_types=[
          pltpu.VMEM((BPW,), jnp.int32),
          pltpu.VMEM_SHARED((ROWS, EMBED), jnp.float32),
          pltpu.VMEM((CHUNK, EMBED), jnp.float32),
          pltpu.VMEM((CHUNK, EMBED), jnp.float32),
          pltpu.SemaphoreType.DMA,
          pltpu.SemaphoreType.DMA,
      ],
  )(_lookup)
  out = run(idx_flat, table)
  return out.reshape(group_idx.shape + (EMBED,))


# speedup vs baseline: 4.6916x; 1.0051x over previous
import functools

import jax
import jax.numpy as jnp
from jax import lax
from jax.experimental import pallas as pl
from jax.experimental.pallas import tpu as pltpu
from jax.experimental.pallas import tpu_sc as plsc

EMBED = 128
ROWS = 17
REP = 4
BATCH = 16384 * 50
NUM_WORKERS = 32
BPW = BATCH // NUM_WORKERS
CHUNK = 128
NCHUNK = BPW // CHUNK


def _lookup(idx_hbm, table_hbm, out_hbm, idx_v, table_sh, s0, s1, s2, s3,
            gsem, wsem):
  sid = lax.axis_index("s")
  wid = sid * 2 + lax.axis_index("c")
  base = wid * BPW

  def load_table():
    for r in range(REP):
      pltpu.sync_copy(table_hbm, table_sh.at[pl.ds(r * ROWS, ROWS)])

  pl.when(sid == 0)(load_table)
  pltpu.sync_copy(idx_hbm.at[pl.ds(base, BPW)], idx_v)

  rep_off = jnp.broadcast_to((sid % REP) * ROWS, (16,)).astype(jnp.int32)

  def adj(j, _):
    idx_v[pl.ds(j * 16, 16)] = idx_v[pl.ds(j * 16, 16)] + rep_off
    return 0

  lax.fori_loop(0, BPW // 16, adj, 0)
  plsc.subcore_barrier()

  bufs = (s0, s1, s2, s3)

  def start_gather(b, i):
    pltpu.async_copy(
        table_sh.at[idx_v.at[pl.ds(i * CHUNK, CHUNK)]], bufs[b], gsem)

  def wait_gather(b):
    pltpu.make_async_copy(
        table_sh.at[idx_v.at[pl.ds(0, CHUNK)]], bufs[b], gsem).wait()

  def start_write(b, i):
    off = base + i * CHUNK
    pltpu.async_copy(bufs[b], out_hbm.at[pl.ds(off, CHUNK)], wsem)

  def drain_write(b):
    pltpu.make_async_copy(bufs[b], out_hbm.at[pl.ds(0, CHUNK)], wsem).wait()

  for j in range(3):
    start_gather(j, j)

  def quad_body(o, _):
    for j in range(4):
      i = 4 * o + j
      wait_gather(j)
      start_write(j, i)
      if j == 0:
        pl.when(i >= 1)(lambda: drain_write(0))
      else:
        drain_write(j - 1)
      nb = (j + 3) % 4
      pl.when(i + 3 < NCHUNK)(lambda i=i, nb=nb: start_gather(nb, i + 3))
    return 0

  lax.fori_loop(0, NCHUNK // 4, quad_body, 0)
  drain_write(3)


def kernel(group_idx, table):
  idx_flat = group_idx.reshape(BATCH)
  mesh = plsc.VectorSubcoreMesh(core_axis_name="c", subcore_axis_name="s")
  run = functools.partial(
      pl.kernel,
      out_type=jax.ShapeDtypeStruct((BATCH, EMBED), jnp.float32),
      mesh=mesh,
      compiler_params=pltpu.CompilerParams(needs_layout_passes=False),
      scratch_types=[
          pltpu.VMEM((BPW,), jnp.int32),
          pltpu.VMEM_SHARED((REP * ROWS, EMBED), jnp.float32),
          pltpu.VMEM((CHUNK, EMBED), jnp.float32),
          pltpu.VMEM((CHUNK, EMBED), jnp.float32),
          pltpu.VMEM((CHUNK, EMBED), jnp.float32),
          pltpu.VMEM((CHUNK, EMBED), jnp.float32),
          pltpu.SemaphoreType.DMA,
          pltpu.SemaphoreType.DMA,
      ],
  )(_lookup)
  out = run(idx_flat, table)
  return out.reshape(group_idx.shape + (EMBED,))


# speedup vs baseline: 9.9665x; 2.1243x over previous
import functools

import jax
import jax.numpy as jnp
from jax import lax
from jax.experimental import pallas as pl
from jax.experimental.pallas import tpu as pltpu
from jax.experimental.pallas import tpu_sc as plsc

EMBED = 128
ROWS = 17
GROUPS = 50
DIM0 = 16384
NUM_WORKERS = 32
D0PW = DIM0 // NUM_WORKERS
N0 = 1
NCHUNK = D0PW // N0


def _lookup(idx_hbm, table_hbm, out_hbm, idx_v, table_sh, s0, s1, s2, s3,
            gsem, wsem):
  sid = lax.axis_index("s")
  wid = sid * 2 + lax.axis_index("c")
  base = wid * D0PW

  pl.when(sid == 0)(lambda: pltpu.sync_copy(table_hbm, table_sh))
  pltpu.sync_copy(idx_hbm.at[pl.ds(base, D0PW)], idx_v)
  plsc.subcore_barrier()

  bufs = (s0, s1, s2, s3)

  def start_gather(b, i):
    pltpu.async_copy(table_sh.at[idx_v.at[i]], bufs[b], gsem)

  def wait_gather(b):
    pltpu.make_async_copy(table_sh.at[idx_v.at[0]], bufs[b], gsem).wait()

  def start_write(b, i):
    pltpu.async_copy(bufs[b], out_hbm.at[base + i], wsem)

  def drain_write(b):
    pltpu.make_async_copy(bufs[b], out_hbm.at[0], wsem).wait()

  for j in range(3):
    start_gather(j, j)

  def quad_body(o, _):
    for j in range(4):
      i = 4 * o + j
      wait_gather(j)
      start_write(j, i)
      if j == 0:
        pl.when(i >= 1)(lambda: drain_write(0))
      else:
        drain_write(j - 1)
      nb = (j + 3) % 4
      pl.when(i + 3 < NCHUNK)(lambda i=i, nb=nb: start_gather(nb, i + 3))
    return 0

  lax.fori_loop(0, NCHUNK // 4, quad_body, 0)
  drain_write(3)


def kernel(group_idx, table):
  mesh = plsc.VectorSubcoreMesh(core_axis_name="c", subcore_axis_name="s")
  run = functools.partial(
      pl.kernel,
      out_type=jax.ShapeDtypeStruct((DIM0, GROUPS, EMBED), jnp.float32),
      mesh=mesh,
      compiler_params=pltpu.CompilerParams(
          needs_layout_passes=False, use_tc_tiling_on_sc=True),
      scratch_types=[
          pltpu.VMEM((D0PW, GROUPS), jnp.int32),
          pltpu.VMEM_SHARED((ROWS, EMBED), jnp.float32),
          pltpu.VMEM((GROUPS, EMBED), jnp.float32),
          pltpu.VMEM((GROUPS, EMBED), jnp.float32),
          pltpu.VMEM((GROUPS, EMBED), jnp.float32),
          pltpu.VMEM((GROUPS, EMBED), jnp.float32),
          pltpu.SemaphoreType.DMA,
          pltpu.SemaphoreType.DMA,
      ],
  )(_lookup)
  return run(group_idx, table)
